# baseline (speedup 1.0000x reference)
import functools

import jax
import jax.numpy as jnp
from jax import lax
from jax.experimental import pallas as pl
from jax.experimental.pallas import tpu as pltpu
from jax.experimental.pallas import tpu_sc as plsc

B, L, K_EDGE = 16, 2048, 30
MIN_B, MAX_B = 0.1, 20.0
NC, NS = 2, 16
HALF = L // 2
EDGE_W = L * K_EDGE * 6
EDGE_GRID = 8


def _sc_shift_body(ca_hbm, nca_hbm, cca_hbm, scal_hbm,
                   ca_out, nca_out, cca_out,
                   rowa, rowb, rowc, scal_v, outa, outb, outc):
    wid = lax.axis_index("s") * NC + lax.axis_index("c")
    b = wid // 2
    h = wid % 2
    pltpu.sync_copy(ca_hbm.at[b], rowa)
    pltpu.sync_copy(nca_hbm.at[b], rowb)
    pltpu.sync_copy(cca_hbm.at[b], rowc)
    pltpu.sync_copy(scal_hbm, scal_v)
    len_b = scal_v[pl.ds(b, 16)][0]
    rs_b = scal_v[pl.ds(b + 16, 16)][0]
    roll = scal_v[pl.ds(32, 16)][0]
    base = h * HALF
    iota = lax.broadcasted_iota(jnp.int32, (16,), 0)

    def chunk(ci, carry):
        i = base + ci * 16 + iota
        jm = lax.rem(lax.rem(i - roll, L) + L, L)
        k = jnp.minimum(jnp.maximum(jm - rs_b, 0), len_b - 1)
        k3 = k * 3
        off = ci * 16
        for c in range(3):
            outa[pl.ds(c * HALF + off, 16)] = plsc.load_gather(rowa, [k3 + c])
            outb[pl.ds(c * HALF + off, 16)] = plsc.load_gather(rowb, [k3 + c])
            outc[pl.ds(c * HALF + off, 16)] = plsc.load_gather(rowc, [k3 + c])
        return carry

    lax.fori_loop(0, HALF // 16, chunk, 0)
    for c in range(3):
        pltpu.sync_copy(outa.at[pl.ds(c * HALF, HALF)],
                        ca_out.at[c, b, pl.ds(base, HALF)])
        pltpu.sync_copy(outb.at[pl.ds(c * HALF, HALF)],
                        nca_out.at[c, b, pl.ds(base, HALF)])
        pltpu.sync_copy(outc.at[pl.ds(c * HALF, HALF)],
                        cca_out.at[c, b, pl.ds(base, HALF)])


@functools.cache
def _sc_shift():
    return pl.kernel(
        _sc_shift_body,
        out_type=(jax.ShapeDtypeStruct((3, B, L), jnp.float32),) * 3,
        mesh=plsc.VectorSubcoreMesh(core_axis_name="c", subcore_axis_name="s",
                                    num_cores=NC, num_subcores=NS),
        compiler_params=pltpu.CompilerParams(needs_layout_passes=False),
        scratch_types=[
            pltpu.VMEM((L * 3,), jnp.float32),
            pltpu.VMEM((L * 3,), jnp.float32),
            pltpu.VMEM((L * 3,), jnp.float32),
            pltpu.VMEM((3 * B,), jnp.int32),
            pltpu.VMEM((3 * HALF,), jnp.float32),
            pltpu.VMEM((3 * HALF,), jnp.float32),
            pltpu.VMEM((3 * HALF,), jnp.float32),
        ],
    )


def _alpha_sigma(t):
    int_beta = t * MIN_B + 0.5 * t * t * (MAX_B - MIN_B)
    alpha = jnp.exp(-0.5 * int_beta)
    sigma = jnp.sqrt(1.0 - jnp.exp(-int_beta))
    return alpha, sigma


def _edges_body(t_ref, noise_ref, out_ref):
    alpha, sigma = _alpha_sigma(t_ref[...])
    par = lax.broadcasted_iota(jnp.int32, out_ref.shape, 1) & 1
    mask = par.astype(jnp.float32)
    out_ref[...] = sigma * noise_ref[...] + alpha * mask


def _backbone_body(t_ref, ca_ref, nc_ref, cc_ref, rot_ref, nca_ref,
                   can_ref, ncn_ref, ccn_ref, ss_ref):
    alpha, sigma = _alpha_sigma(t_ref[...])
    ss_ref[...] = 1.0 / sigma
    vx, vy, vz = rot_ref[0], rot_ref[1], rot_ref[2]
    theta = jnp.sqrt(vx * vx + vy * vy + vz * vz)
    safe = jnp.where(theta < 1e-8, 1.0, theta)
    inv = 1.0 / safe
    kx, ky, kz = vx * inv, vy * inv, vz * inv
    sn = jnp.sin(theta)
    c1 = 1.0 - jnp.cos(theta)

    def rodrigues(ref, oref):
        x, y, z = ref[0], ref[1], ref[2]
        cx = ky * z - kz * y
        cy = kz * x - kx * z
        cz = kx * y - ky * x
        dx = ky * cz - kz * cy
        dy = kz * cx - kx * cz
        dz = kx * cy - ky * cx
        oref[0] = x + sn * cx + c1 * dx
        oref[1] = y + sn * cy + c1 * dy
        oref[2] = z + sn * cz + c1 * dz

    rodrigues(nc_ref, ncn_ref)
    rodrigues(cc_ref, ccn_ref)
    for c in range(3):
        can_ref[c] = alpha * ca_ref[c] + sigma * nca_ref[c]


def kernel(ca, n_ca, c_ca, lengths, randstart, randroll, t_vec, rot_vec,
           noise_ca, noise_edges):
    scal = jnp.concatenate([lengths.astype(jnp.int32),
                            randstart.astype(jnp.int32),
                            jnp.full((B,), randroll, dtype=jnp.int32)])
    t_col = t_vec.reshape(B, 1)

    ca_s, nc_s, cc_s = _sc_shift()(ca.reshape(B, L * 3), n_ca.reshape(B, L * 3),
                                   c_ca.reshape(B, L * 3), scal)

    noise_flat = noise_edges.reshape(B, EDGE_W)
    w = EDGE_W // EDGE_GRID
    edges_flat = pl.pallas_call(
        _edges_body,
        grid=(EDGE_GRID,),
        in_specs=[pl.BlockSpec((B, 1), lambda g: (0, 0)),
                  pl.BlockSpec((B, w), lambda g: (0, g))],
        out_specs=pl.BlockSpec((B, w), lambda g: (0, g)),
        out_shape=jax.ShapeDtypeStruct((B, EDGE_W), jnp.float32),
    )(t_col, noise_flat)

    rot3 = rot_vec.reshape(B, L, 3).transpose(2, 0, 1)
    noise3 = noise_ca.transpose(2, 0, 1)
    plane = jax.ShapeDtypeStruct((3, B, L), jnp.float32)
    can, ncn, ccn, ss = pl.pallas_call(
        _backbone_body,
        out_shape=(plane, plane, plane,
                   jax.ShapeDtypeStruct((B, 1), jnp.float32)),
    )(t_col, ca_s, nc_s, cc_s, rot3, noise3)

    ca_noised = can.transpose(1, 2, 0)
    nc_noised = ncn.transpose(1, 2, 0)
    cc_noised = ccn.transpose(1, 2, 0)
    score_scales = ss.reshape(B)
    edges_noised = edges_flat.reshape(B, L, K_EDGE, 3, 2)
    return (ca_noised, nc_noised, cc_noised, t_vec, score_scales, edges_noised)
